# baseline (speedup 1.0000x reference)
import functools

import jax
import jax.numpy as jnp
from jax import lax
from jax.experimental import pallas as pl
from jax.experimental.pallas import tpu as pltpu
from jax.experimental.pallas import tpu_sc as plsc

_DIM = (192, 192, 192)
_C = 4
_FLAT = _DIM[0] * _DIM[1] * _DIM[2]
_NW = 32
_NSLAB = 16
_SLAB = _FLAT // _NSLAB
_CH = 2048
_S = 2048
_SB = _S + 16
_FILL = 16384
_DUMP = _FLAT + 1

_mesh = plsc.VectorSubcoreMesh(core_axis_name="c", subcore_axis_name="s")


def _worker_id():
    return lax.axis_index("s") * 2 + lax.axis_index("c")


def _settle(stg_idx, stg_rid, wback, w_hbm):
    dump16 = jnp.full((16,), _DUMP, jnp.int32)
    for _ in range(2):
        pltpu.sync_copy(w_hbm.at[stg_idx], wback)

        def fix(g, _):
            p = g * 16
            v = stg_idx[pl.ds(p, 16)]
            r = stg_rid[pl.ds(p, 16)]
            wb = wback[pl.ds(p, 16)]
            stg_idx[pl.ds(p, 16)] = jnp.where(wb < r, v, dump16)
            return _

        lax.fori_loop(0, _SB // 16, fix, jnp.int32(0))
        pltpu.sync_copy(stg_rid, w_hbm.at[stg_idx])


def _scan_array(flat_hbm, w_hbm, lo, nchunk, vchunk, stg_idx, stg_rid, wback):
    iota = lax.iota(jnp.int32, 16)
    dump16 = jnp.full((16,), _DUMP, jnp.int32)

    def chunk_body(c, off):
        pltpu.sync_copy(flat_hbm.at[pl.ds(c * _CH, _CH)], vchunk)

        def grp(j, off):
            v = vchunk[pl.ds(j * 16, 16)]
            rid = (c * _CH + j * 16) + iota
            m = (v - lo).astype(jnp.uint32) < jnp.uint32(_SLAB)
            cnt = jnp.max(plsc.all_reduce_population_count(m))
            plsc.store_compressed(stg_idx.at[pl.ds(off, 16)], v, mask=m)
            plsc.store_compressed(stg_rid.at[pl.ds(off, 16)], rid, mask=m)
            off = off + cnt

            @pl.when(off >= _S)
            def _flush():
                tail_pos = _S + iota
                plsc.store_scatter(stg_idx, [tail_pos], dump16, mask=tail_pos >= off)
                pltpu.sync_copy(stg_rid, w_hbm.at[stg_idx])
                stg_idx[pl.ds(0, 16)] = stg_idx[pl.ds(_S, 16)]
                stg_rid[pl.ds(0, 16)] = stg_rid[pl.ds(_S, 16)]
                _settle(stg_idx, stg_rid, wback, w_hbm)

            return jnp.where(off >= _S, off - _S, off)

        return lax.fori_loop(0, _CH // 16, grp, off)

    off = lax.fori_loop(0, nchunk, chunk_body, jnp.int32(0))

    def pad_body(p):
        stg_idx[pl.ds(jnp.minimum(p, _S), 16)] = dump16
        return p + 16

    lax.while_loop(lambda p: p < _SB, pad_body, off)
    pltpu.sync_copy(stg_rid, w_hbm.at[stg_idx])
    _settle(stg_idx, stg_rid, wback, w_hbm)


def _winner_kernel(nchunk):
    @functools.partial(
        pl.kernel,
        out_type=[
            jax.ShapeDtypeStruct((_FLAT + 8,), jnp.int32),
            jax.ShapeDtypeStruct((_FLAT + 8,), jnp.int32),
        ],
        mesh=_mesh,
        compiler_params=pltpu.CompilerParams(needs_layout_passes=False),
        scratch_types=[
            pltpu.VMEM((_CH,), jnp.int32),
            pltpu.VMEM((_SB,), jnp.int32),
            pltpu.VMEM((_SB,), jnp.int32),
            pltpu.VMEM((_FILL,), jnp.int32),
            pltpu.VMEM((_SB,), jnp.int32),
        ],
    )
    def kern(cur_flat, glb_flat, fill_src, w_cur, w_glb,
             vchunk, stg_idx, stg_rid, fillbuf, wback):
        wid = _worker_id()

        @pl.when(wid < _NSLAB)
        def _cur_side():
            _scan_array(cur_flat, w_cur, wid * _SLAB, nchunk,
                        vchunk, stg_idx, stg_rid, wback)

            @pl.when(wid == _NSLAB - 1)
            def _sentinel():
                pltpu.sync_copy(fill_src.at[pl.ds(0, 8)], fillbuf.at[pl.ds(0, 8)])
                pltpu.sync_copy(fillbuf.at[pl.ds(0, 8)], w_cur.at[pl.ds(_FLAT, 8)])

        @pl.when(wid >= _NSLAB)
        def _glb_side():
            sid = wid - _NSLAB
            lo = sid * _SLAB
            pltpu.sync_copy(fill_src, fillbuf)

            def fill_body(k, _):
                pltpu.sync_copy(fillbuf, w_glb.at[pl.ds(lo + k * _FILL, _FILL)])
                return _

            lax.fori_loop(0, _SLAB // _FILL, fill_body, jnp.int32(0))

            @pl.when(sid == _NSLAB - 1)
            def _sentinel():
                pltpu.sync_copy(fillbuf.at[pl.ds(0, 8)], w_glb.at[pl.ds(_FLAT, 8)])

            _scan_array(glb_flat, w_glb, lo, nchunk,
                        vchunk, stg_idx, stg_rid, wback)

    return kern


def _gather_kernel(nchunk):
    n_pad = nchunk * _CH

    @functools.partial(
        pl.kernel,
        out_type=[
            jax.ShapeDtypeStruct((n_pad * _C,), jnp.float32),
            jax.ShapeDtypeStruct((n_pad * _C,), jnp.float32),
        ],
        mesh=_mesh,
        compiler_params=pltpu.CompilerParams(needs_layout_passes=False),
        scratch_types=[
            pltpu.VMEM((_CH,), jnp.int32),
            pltpu.VMEM((_CH,), jnp.int32),
            pltpu.VMEM((_CH,), jnp.int32),
            pltpu.VMEM((_CH * _C,), jnp.int32),
            pltpu.VMEM((_CH * _C,), jnp.int32),
            pltpu.VMEM((_CH * _C,), jnp.float32),
            pltpu.VMEM((_CH * _C,), jnp.float32),
            pltpu.SemaphoreType.DMA,
            pltpu.SemaphoreType.DMA,
        ],
    )
    def kern(cur_flat, w_cur, w_glb, cur_vals, glb_vals, out_c, out_g,
             vflat, vrc, vrg, eic, eig, vc, vg, sem1, sem2):
        wid = _worker_id()
        rounds = (nchunk + _NW - 1) // _NW
        iota = lax.iota(jnp.int32, 16)

        def body(k, _):
            c = wid + _NW * k

            @pl.when(c < nchunk)
            def _do():
                base = c * _CH
                pltpu.sync_copy(cur_flat.at[pl.ds(base, _CH)], vflat)
                h1 = pltpu.async_copy(w_cur.at[vflat], vrc, sem1)
                h2 = pltpu.async_copy(w_glb.at[vflat], vrg, sem2)
                h1.wait()
                h2.wait()

                def mk(j, _):
                    pos = j * 16 + iota
                    rc = plsc.load_gather(vrc, [pos]) * _C
                    rg = plsc.load_gather(vrg, [pos]) * _C
                    for ch in range(_C):
                        plsc.store_scatter(eic, [pos * _C + ch], rc + ch)
                        plsc.store_scatter(eig, [pos * _C + ch], rg + ch)
                    return _

                lax.fori_loop(0, _CH // 16, mk, jnp.int32(0))
                h3 = pltpu.async_copy(cur_vals.at[eic], vc, sem1)
                h4 = pltpu.async_copy(glb_vals.at[eig], vg, sem2)
                h3.wait()
                h4.wait()
                pltpu.sync_copy(vc, out_c.at[pl.ds(base * _C, _CH * _C)])
                pltpu.sync_copy(vg, out_g.at[pl.ds(base * _C, _CH * _C)])

            return _

        lax.fori_loop(0, rounds, body, jnp.int32(0))

    return kern


def kernel(current_coords, current_values, global_coords, global_values, relative_origin):
    n_cur = current_coords.shape[0]
    n_glb = global_coords.shape[0]
    dim = jnp.array(_DIM, dtype=jnp.int32)

    cur_flat = (current_coords[:, 0] * _DIM[1] + current_coords[:, 1]) * _DIM[2] \
        + current_coords[:, 2]
    gc = global_coords - relative_origin[None, :]
    bounds_ok = jnp.all((gc < dim[None, :]) & (gc >= 0), axis=-1)
    gc_cl = jnp.clip(gc, 0, dim[None, :] - 1)
    g_flat = jnp.where(
        bounds_ok,
        (gc_cl[:, 0] * _DIM[1] + gc_cl[:, 1]) * _DIM[2] + gc_cl[:, 2],
        _FLAT,
    )

    nchunk = -(-max(n_cur, n_glb) // _CH)
    n_pad = nchunk * _CH
    cur_flat_p = jnp.full((n_pad,), _FLAT, jnp.int32).at[:n_cur].set(cur_flat)
    g_flat_p = jnp.full((n_pad,), _FLAT, jnp.int32).at[:n_glb].set(g_flat)

    cur_ext = jnp.concatenate(
        [current_values, jnp.ones((8, _C), jnp.float32)], axis=0).reshape(-1)
    glb_ext = jnp.concatenate(
        [global_values, jnp.ones((8, _C), jnp.float32)], axis=0).reshape(-1)
    fill_src = jnp.full((_FILL,), n_glb, jnp.int32)

    w_cur, w_glb = _winner_kernel(nchunk)(cur_flat_p, g_flat_p, fill_src)
    out_c, out_g = _gather_kernel(nchunk)(
        cur_flat_p, w_cur, w_glb, cur_ext, glb_ext)

    out_c = out_c.reshape(n_pad, _C)[:n_cur]
    out_g = out_g.reshape(n_pad, _C)[:n_cur]
    return jnp.concatenate([out_c, out_g], axis=-1)
